# baseline (speedup 1.0000x reference)
import jax
import jax.numpy as jnp
from jax.experimental import pallas as pl
from jax.experimental.pallas import tpu as pltpu


_N = 4096
_F = 128
_BMS = 256
_NBS = _N // _BMS
_BMR = 1024
_NBR = _N // _BMR
_BMG = 1024
_NBG = _N // _BMG


def _encdec_kernel(adj_ref, x_ref, w1_ref, w2_ref, w3_ref, w4_ref, w5_ref,
                   w6_ref, zig_ref, zigb_ref, zhat_ref, zhatb_ref,
                   adj16_ref, feat_ref, supa_ref, supb_ref):
    i = pl.program_id(0)

    @pl.when(i == 0)
    def _():
        supa_ref[...] = jnp.tanh(x_ref[...] @ w1_ref[...]).astype(jnp.bfloat16)

    a = adj_ref[...].astype(jnp.bfloat16)
    rows = pl.ds(i * _BMS, _BMS)
    adj16_ref[rows, :] = a
    f1 = jax.lax.dot_general(
        a, supa_ref[...], (((1,), (0,)), ((), ())),
        preferred_element_type=jnp.float32)
    feat_ref[rows, :] = f1
    supb_ref[rows, :64] = jnp.tanh(f1 @ w2_ref[...]).astype(jnp.bfloat16)

    @pl.when(i == _NBS - 1)
    def _():
        def layer(src_sup_ref, dst_ref, post):
            sup = src_sup_ref[...]
            for j in range(_NBR):
                r = pl.ds(j * _BMR, _BMR)
                f = jax.lax.dot_general(
                    adj16_ref[r, :], sup,
                    (((1,), (0,)), ((), ())),
                    preferred_element_type=jnp.float32)
                dst_ref[r, :] = f
                post(r, f)

        def next_sup(w_ref, active, dst_sup_ref):
            fin, fout = w_ref.shape

            def post(r, f):
                s = f[:, :fin] @ w_ref[...]
                if active:
                    s = jnp.tanh(s)
                dst_sup_ref[r, :fout] = s.astype(jnp.bfloat16)

            return post

        layer(supb_ref, feat_ref, next_sup(w3_ref, False, supa_ref))

        def post3(r, f):
            zig_ref[r, :] = f[:, :32]
            zigb_ref[r, :] = f.astype(jnp.bfloat16)
            supb_ref[r, :64] = jnp.tanh(f[:, :32] @ w4_ref[...]).astype(
                jnp.bfloat16)

        layer(supa_ref, feat_ref, post3)
        layer(supb_ref, feat_ref, next_sup(w5_ref, True, supa_ref))
        layer(supa_ref, feat_ref, next_sup(w6_ref, True, supb_ref))

        def post6(r, f):
            zhatb_ref[r, :] = f.astype(jnp.bfloat16)

        layer(supb_ref, zhat_ref, post6)


def _gram_kernel(z_ref, zfull_ref, out_ref):
    s = jax.lax.dot_general(
        z_ref[...], zfull_ref[...], (((1,), (1,)), ((), ())),
        preferred_element_type=jnp.float32)
    out_ref[...] = 0.5 * jnp.tanh(0.5 * s) + 0.5


def _gram(zb):
    n, f = zb.shape
    return pl.pallas_call(
        _gram_kernel,
        grid=(_NBG,),
        in_specs=[
            pl.BlockSpec((_BMG, f), lambda i: (i, 0)),
            pl.BlockSpec((n, f), lambda i: (0, 0)),
        ],
        out_specs=pl.BlockSpec((_BMG, n), lambda i: (i, 0)),
        out_shape=jax.ShapeDtypeStruct((n, n), jnp.float32),
    )(zb, zb)


def kernel(x, adj, W1, W2, W3, W4, W5, W6):
    w3 = jnp.pad(W3, ((0, 0), (0, _F - W3.shape[1])))
    z_igae, zigb, z_hat, zhatb = pl.pallas_call(
        _encdec_kernel,
        grid=(_NBS,),
        in_specs=[
            pl.BlockSpec((_BMS, _N), lambda i: (i, 0)),
            pl.BlockSpec((_N, _F), lambda i: (0, 0)),
            pl.BlockSpec(W1.shape, lambda i: (0, 0)),
            pl.BlockSpec(W2.shape, lambda i: (0, 0)),
            pl.BlockSpec((W3.shape[0], _F), lambda i: (0, 0)),
            pl.BlockSpec(W4.shape, lambda i: (0, 0)),
            pl.BlockSpec(W5.shape, lambda i: (0, 0)),
            pl.BlockSpec(W6.shape, lambda i: (0, 0)),
        ],
        out_specs=[
            pl.BlockSpec((_N, 32), lambda i: (0, 0)),
            pl.BlockSpec((_N, _F), lambda i: (0, 0)),
            pl.BlockSpec((_N, _F), lambda i: (0, 0)),
            pl.BlockSpec((_N, _F), lambda i: (0, 0)),
        ],
        out_shape=[
            jax.ShapeDtypeStruct((_N, 32), jnp.float32),
            jax.ShapeDtypeStruct((_N, _F), jnp.bfloat16),
            jax.ShapeDtypeStruct((_N, _F), jnp.float32),
            jax.ShapeDtypeStruct((_N, _F), jnp.bfloat16),
        ],
        scratch_shapes=[
            pltpu.VMEM((_N, _N), jnp.bfloat16),
            pltpu.VMEM((_N, _F), jnp.float32),
            pltpu.VMEM((_N, _F), jnp.bfloat16),
            pltpu.VMEM((_N, _F), jnp.bfloat16),
        ],
    )(adj, x, W1, W2, w3, W4, W5, W6)
    z_igae_adj = _gram(zigb)
    z_hat_adj = _gram(zhatb)
    return (z_igae, z_igae_adj, z_hat, z_hat_adj)


# speedup vs baseline: 1.2623x; 1.2623x over previous
import jax
import jax.numpy as jnp
from jax.experimental import pallas as pl
from jax.experimental.pallas import tpu as pltpu


_N = 4096
_F = 128
_BMS = 256
_NBS = _N // _BMS
_BMR = 1024
_NBR = _N // _BMR
_BMG = 1024
_NBG = _N // _BMG


def _encdec_kernel(adj_ref, x_ref, w1_ref, w2_ref, w3_ref, w4_ref, w5_ref,
                   w6_ref, zig_ref, zigb_ref, zhat_ref, zhatb_ref,
                   adj16_ref, feat_ref, zigp_ref, sup_ref):
    i = pl.program_id(0)

    @pl.when(i == 0)
    def _():
        sup_ref[...] = jnp.tanh(x_ref[...] @ w1_ref[...]).astype(jnp.bfloat16)

    a = adj_ref[...].astype(jnp.bfloat16)
    rows = pl.ds(i * _BMS, _BMS)
    adj16_ref[rows, :] = a
    feat_ref[rows, :] = jax.lax.dot_general(
        a, sup_ref[...], (((1,), (0,)), ((), ())),
        preferred_element_type=jnp.float32)

    @pl.when(i == _NBS - 1)
    def _():
        def spmm(dst_ref):
            sup = sup_ref[...]
            for j in range(_NBR):
                r = pl.ds(j * _BMR, _BMR)
                dst_ref[r, :] = jax.lax.dot_general(
                    adj16_ref[r, :], sup,
                    (((1,), (0,)), ((), ())),
                    preferred_element_type=jnp.float32)

        def support(src_ref, w_ref, active):
            fin = w_ref.shape[0]
            s = src_ref[:, :fin] @ w_ref[...]
            if active:
                s = jnp.tanh(s)
            sup_ref[:, :s.shape[1]] = s.astype(jnp.bfloat16)

        support(feat_ref, w2_ref, True)
        spmm(feat_ref)
        support(feat_ref, w3_ref, False)
        spmm(zigp_ref)
        zig_ref[...] = zigp_ref[:, :32]
        zigb_ref[...] = zigp_ref[...].astype(jnp.bfloat16)
        support(zigp_ref, w4_ref, True)
        spmm(feat_ref)
        support(feat_ref, w5_ref, True)
        spmm(feat_ref)
        support(feat_ref, w6_ref, True)
        spmm(zhat_ref)
        zhatb_ref[...] = zhat_ref[...].astype(jnp.bfloat16)


def _gram_kernel(z_ref, zfull_ref, out_ref):
    s = jax.lax.dot_general(
        z_ref[...], zfull_ref[...], (((1,), (1,)), ((), ())),
        preferred_element_type=jnp.float32)
    out_ref[...] = 0.5 * jnp.tanh(0.5 * s) + 0.5


def _gram(zb):
    n, f = zb.shape
    return pl.pallas_call(
        _gram_kernel,
        grid=(_NBG,),
        in_specs=[
            pl.BlockSpec((_BMG, f), lambda i: (i, 0)),
            pl.BlockSpec((n, f), lambda i: (0, 0)),
        ],
        out_specs=pl.BlockSpec((_BMG, n), lambda i: (i, 0)),
        out_shape=jax.ShapeDtypeStruct((n, n), jnp.float32),
    )(zb, zb)


def kernel(x, adj, W1, W2, W3, W4, W5, W6):
    w3 = jnp.pad(W3, ((0, 0), (0, _F - W3.shape[1])))
    z_igae, zigb, z_hat, zhatb = pl.pallas_call(
        _encdec_kernel,
        grid=(_NBS,),
        in_specs=[
            pl.BlockSpec((_BMS, _N), lambda i: (i, 0)),
            pl.BlockSpec((_N, _F), lambda i: (0, 0)),
            pl.BlockSpec(W1.shape, lambda i: (0, 0)),
            pl.BlockSpec(W2.shape, lambda i: (0, 0)),
            pl.BlockSpec((W3.shape[0], _F), lambda i: (0, 0)),
            pl.BlockSpec(W4.shape, lambda i: (0, 0)),
            pl.BlockSpec(W5.shape, lambda i: (0, 0)),
            pl.BlockSpec(W6.shape, lambda i: (0, 0)),
        ],
        out_specs=[
            pl.BlockSpec((_N, 32), lambda i: (0, 0)),
            pl.BlockSpec((_N, _F), lambda i: (0, 0)),
            pl.BlockSpec((_N, _F), lambda i: (0, 0)),
            pl.BlockSpec((_N, _F), lambda i: (0, 0)),
        ],
        out_shape=[
            jax.ShapeDtypeStruct((_N, 32), jnp.float32),
            jax.ShapeDtypeStruct((_N, _F), jnp.bfloat16),
            jax.ShapeDtypeStruct((_N, _F), jnp.float32),
            jax.ShapeDtypeStruct((_N, _F), jnp.bfloat16),
        ],
        scratch_shapes=[
            pltpu.VMEM((_N, _N), jnp.bfloat16),
            pltpu.VMEM((_N, _F), jnp.float32),
            pltpu.VMEM((_N, _F), jnp.float32),
            pltpu.VMEM((_N, _F), jnp.bfloat16),
        ],
    )(adj, x, W1, W2, w3, W4, W5, W6)
    z_igae_adj = _gram(zigb)
    z_hat_adj = _gram(zhatb)
    return (z_igae, z_igae_adj, z_hat, z_hat_adj)


# speedup vs baseline: 1.3776x; 1.0914x over previous
import jax
import jax.numpy as jnp
from jax.experimental import pallas as pl
from jax.experimental.pallas import tpu as pltpu


_N = 4096
_F = 128
_BMS = 256
_NBS = _N // _BMS
_BMR = 1024
_NBR = _N // _BMR
_BMG = 512
_NBG = _N // _BMG


def _encdec_kernel(adj_ref, x_ref, w1_ref, w2_ref, w3_ref, w4_ref, w5_ref,
                   w6_ref, zig_ref, zigb_ref, zhat_ref, zhatb_ref,
                   adj16_ref, feat_ref, zigp_ref, sup_ref):
    i = pl.program_id(0)

    @pl.when(i == 0)
    def _():
        sup_ref[...] = jnp.tanh(x_ref[...] @ w1_ref[...]).astype(jnp.bfloat16)

    a = adj_ref[...].astype(jnp.bfloat16)
    rows = pl.ds(i * _BMS, _BMS)
    adj16_ref[rows, :] = a
    feat_ref[rows, :] = jax.lax.dot_general(
        a, sup_ref[...], (((1,), (0,)), ((), ())),
        preferred_element_type=jnp.float32)

    @pl.when(i == _NBS - 1)
    def _():
        def spmm(dst_ref):
            sup = sup_ref[...]
            for j in range(_NBR):
                r = pl.ds(j * _BMR, _BMR)
                dst_ref[r, :] = jax.lax.dot_general(
                    adj16_ref[r, :], sup,
                    (((1,), (0,)), ((), ())),
                    preferred_element_type=jnp.float32)

        def support(src_ref, w_ref, active):
            fin = w_ref.shape[0]
            s = src_ref[:, :fin] @ w_ref[...]
            if active:
                s = jnp.tanh(s)
            sup_ref[:, :s.shape[1]] = s.astype(jnp.bfloat16)

        support(feat_ref, w2_ref, True)
        spmm(feat_ref)
        support(feat_ref, w3_ref, False)
        spmm(zigp_ref)
        zig_ref[...] = zigp_ref[:, :32]
        zigb_ref[...] = zigp_ref[...].astype(jnp.bfloat16)
        support(zigp_ref, w4_ref, True)
        spmm(feat_ref)
        support(feat_ref, w5_ref, True)
        spmm(feat_ref)
        support(feat_ref, w6_ref, True)
        spmm(zhat_ref)
        zhatb_ref[...] = zhat_ref[...].astype(jnp.bfloat16)


def _grams_kernel(zigb_ref, zhatb_ref, g1_ref, g2_ref):
    i = pl.program_id(0)

    def block(src_ref, j, out_ref):
        s = jax.lax.dot_general(
            src_ref[pl.ds(j * _BMG, _BMG), :], src_ref[...],
            (((1,), (1,)), ((), ())),
            preferred_element_type=jnp.float32)
        out_ref[...] = 0.5 * jnp.tanh(0.5 * s) + 0.5

    @pl.when(i < _NBG)
    def _():
        block(zigb_ref, i, g1_ref)

    @pl.when(i >= _NBG)
    def _():
        block(zhatb_ref, i - _NBG, g2_ref)


def _grams(zigb, zhatb):
    return pl.pallas_call(
        _grams_kernel,
        grid=(2 * _NBG,),
        in_specs=[
            pl.BlockSpec((_N, _F), lambda i: (0, 0)),
            pl.BlockSpec((_N, _F), lambda i: (0, 0)),
        ],
        out_specs=[
            pl.BlockSpec((_BMG, _N),
                         lambda i: (jnp.minimum(i, _NBG - 1), 0)),
            pl.BlockSpec((_BMG, _N),
                         lambda i: (jnp.maximum(i - _NBG, 0), 0)),
        ],
        out_shape=[
            jax.ShapeDtypeStruct((_N, _N), jnp.float32),
            jax.ShapeDtypeStruct((_N, _N), jnp.float32),
        ],
    )(zigb, zhatb)


def kernel(x, adj, W1, W2, W3, W4, W5, W6):
    w3 = jnp.pad(W3, ((0, 0), (0, _F - W3.shape[1])))
    z_igae, zigb, z_hat, zhatb = pl.pallas_call(
        _encdec_kernel,
        grid=(_NBS,),
        in_specs=[
            pl.BlockSpec((_BMS, _N), lambda i: (i, 0)),
            pl.BlockSpec((_N, _F), lambda i: (0, 0)),
            pl.BlockSpec(W1.shape, lambda i: (0, 0)),
            pl.BlockSpec(W2.shape, lambda i: (0, 0)),
            pl.BlockSpec((W3.shape[0], _F), lambda i: (0, 0)),
            pl.BlockSpec(W4.shape, lambda i: (0, 0)),
            pl.BlockSpec(W5.shape, lambda i: (0, 0)),
            pl.BlockSpec(W6.shape, lambda i: (0, 0)),
        ],
        out_specs=[
            pl.BlockSpec((_N, 32), lambda i: (0, 0)),
            pl.BlockSpec((_N, _F), lambda i: (0, 0)),
            pl.BlockSpec((_N, _F), lambda i: (0, 0)),
            pl.BlockSpec((_N, _F), lambda i: (0, 0)),
        ],
        out_shape=[
            jax.ShapeDtypeStruct((_N, 32), jnp.float32),
            jax.ShapeDtypeStruct((_N, _F), jnp.bfloat16),
            jax.ShapeDtypeStruct((_N, _F), jnp.float32),
            jax.ShapeDtypeStruct((_N, _F), jnp.bfloat16),
        ],
        scratch_shapes=[
            pltpu.VMEM((_N, _N), jnp.bfloat16),
            pltpu.VMEM((_N, _F), jnp.float32),
            pltpu.VMEM((_N, _F), jnp.float32),
            pltpu.VMEM((_N, _F), jnp.bfloat16),
        ],
    )(adj, x, W1, W2, w3, W4, W5, W6)
    z_igae_adj, z_hat_adj = _grams(zigb, zhatb)
    return (z_igae, z_igae_adj, z_hat, z_hat_adj)
